# Initial kernel scaffold; baseline (speedup 1.0000x reference)
import jax
import jax.numpy as jnp
from jax.experimental import pallas as pl


def kernel(x, w1, b1, g1, be1, w2, b2, g2, be2, w3, b3, g3, be3, w4, b4, codebook):
    raise NotImplementedError("write your pallas kernel here")


# speedup vs baseline: 1.2303x; 1.2303x over previous
import jax
import jax.numpy as jnp
from jax.experimental import pallas as pl

BATCH = 64
T = 1024
D = 256
K = 1024
EPS = 1e-5
F32 = jnp.float32
BF16 = jnp.bfloat16


def _conv1d(x, w, b, pad):
    out = jax.lax.conv_general_dilated(
        x, w, window_strides=(1,), padding=[(pad, pad)],
        dimension_numbers=('NCH', 'OIH', 'NCH'))
    return out + b[None, :, None]


def _batchnorm(x, gamma, beta):
    mean = jnp.mean(x, axis=(0, 2), keepdims=True)
    var = jnp.var(x, axis=(0, 2), keepdims=True)
    xn = (x - mean) / jnp.sqrt(var + EPS)
    return xn * gamma[None, :, None] + beta[None, :, None]


def _vq_body(ze_in_ref, cb_ref, cb2_ref, zq_ref):
    ze = ze_in_ref[0]
    cb = cb_ref[...]
    cross = jax.lax.dot_general(ze.astype(BF16), cb.astype(BF16),
                                (((1,), (1,)), ((), ())),
                                preferred_element_type=F32)
    scores = cb2_ref[...] - 2.0 * cross
    m = jnp.min(scores, axis=1, keepdims=True)
    colk = jax.lax.broadcasted_iota(jnp.int32, (T, K), 1)
    ids = jnp.min(jnp.where(scores <= m, colk, K), axis=1)
    onehot = (colk == ids[:, None]).astype(F32)
    zq = jax.lax.dot_general(onehot, cb, (((1,), (0,)), ((), ())),
                             preferred_element_type=F32,
                             precision=jax.lax.Precision.HIGHEST)
    zq_ref[0] = ze + (zq - ze)


def kernel(x, w1, b1, g1, be1, w2, b2, g2, be2, w3, b3, g3, be3, w4, b4,
           codebook):
    h = jnp.swapaxes(x, -1, -2)
    h = jax.nn.relu(_batchnorm(_conv1d(h, w1, b1, 2), g1, be1))
    h = jax.nn.relu(_batchnorm(_conv1d(h, w2, b2, 1), g2, be2))
    h = jax.nn.relu(_batchnorm(_conv1d(h, w3, b3, 2), g3, be3))
    z_e = jnp.swapaxes(_conv1d(h, w4, b4, 1), -1, -2)

    cb2 = jnp.sum(codebook * codebook, axis=-1).reshape(1, K)
    zq = pl.pallas_call(
        _vq_body,
        grid=(BATCH,),
        in_specs=[pl.BlockSpec((1, T, D), lambda i: (i, 0, 0)),
                  pl.BlockSpec((K, D), lambda i: (0, 0)),
                  pl.BlockSpec((1, K), lambda i: (0, 0))],
        out_specs=pl.BlockSpec((1, T, D), lambda i: (i, 0, 0)),
        out_shape=jax.ShapeDtypeStruct((BATCH, T, D), F32),
    )(z_e, codebook, cb2)
    return z_e, zq
